# baseline (speedup 1.0000x reference)
import jax
import jax.numpy as jnp
from jax import lax
from jax.experimental import pallas as pl
from jax.experimental.pallas import tpu as pltpu
from jax.experimental.pallas import tpu_sc as plsc

D = 64
SEQ = 200
NB = 1024

_info = plsc.get_sparse_core_info()
_NC, _NS, _L = _info.num_cores, _info.num_subcores, _info.num_lanes
NW = _NC * _NS
BPW = NB // NW
SPC = 4
CHUNK = SPC * BPW
NCHUNK = SEQ // SPC
NBUF = 2
NG = NCHUNK // NBUF


def _sc_body(x_hbm, oidx_hbm, cos_hbm, sin_hbm, table_hbm, out_hbm,
             idx_v, oidx_v, cos_v, sin_v,
             in0, in1, out0, out1, si0, si1, so0, so1):
    wid = lax.axis_index("s") * _NC + lax.axis_index("c")
    pltpu.sync_copy(x_hbm.at[wid], idx_v)
    pltpu.sync_copy(oidx_hbm.at[wid], oidx_v)
    pltpu.sync_copy(cos_hbm, cos_v)
    pltpu.sync_copy(sin_hbm, sin_v)
    perm = lax.iota(jnp.int32, _L) ^ 1

    bufs = ((in0, out0, si0, so0), (in1, out1, si1, so1))

    def gather(c, b):
        pltpu.async_copy(table_hbm.at[idx_v.at[c]], bufs[b][0], bufs[b][2])

    def gather_wait(c, b):
        pltpu.make_async_copy(table_hbm.at[idx_v.at[c]], bufs[b][0],
                              bufs[b][2]).wait()

    def wb(c, b):
        pltpu.async_copy(bufs[b][1], out_hbm.at[oidx_v.at[c]], bufs[b][3])

    def wb_wait(c, b):
        pltpu.make_async_copy(bufs[b][1], out_hbm.at[oidx_v.at[c]],
                              bufs[b][3]).wait()

    for b in range(NBUF):
        gather(b, b)

    def group_body(g, carry):
        for b in range(NBUF):
            c = g * NBUF + b
            in_v, out_v = bufs[b][0], bufs[b][1]
            gather_wait(c, b)

            @pl.when(g >= 1)
            def _():
                wb_wait(c - NBUF, b)

            for j in range(SPC):
                s = c * SPC + j
                cc = [cos_v[s, pl.ds(_L * k, _L)] for k in range(D // _L)]
                ss = [sin_v[s, pl.ds(_L * k, _L)] for k in range(D // _L)]

                def row_body(i, carry2, j=j, cc=cc, ss=ss):
                    r = j * BPW + i
                    for k in range(D // _L):
                        xv = in_v[r, pl.ds(_L * k, _L)]
                        sw = jnp.take_along_axis(xv, perm, axis=0,
                                                 mode="promise_in_bounds")
                        out_v[r, pl.ds(_L * k, _L)] = xv * cc[k] + sw * ss[k]
                    return carry2

                lax.fori_loop(0, BPW, row_body, 0, unroll=2)

            wb(c, b)

            @pl.when(c + NBUF < NCHUNK)
            def _():
                gather(c + NBUF, b)
        return carry

    lax.fori_loop(0, NG, group_body, 0)
    for b in range(NBUF):
        wb_wait(NCHUNK - NBUF + b, b)


def kernel(x, table):
    nb, seq = x.shape
    x4 = x.astype(jnp.int32).reshape(NW, BPW, NCHUNK, SPC)
    x_r = x4.transpose(0, 2, 3, 1).reshape(NW, NCHUNK, CHUNK)
    w_ = jnp.arange(NW, dtype=jnp.int32)[:, None, None, None]
    c_ = jnp.arange(NCHUNK, dtype=jnp.int32)[None, :, None, None]
    j_ = jnp.arange(SPC, dtype=jnp.int32)[None, None, :, None]
    b_ = jnp.arange(BPW, dtype=jnp.int32)[None, None, None, :]
    oidx = ((w_ * BPW + b_) * SEQ + c_ * SPC + j_).reshape(NW, NCHUNK, CHUNK)

    inv_freq = 1.0 / (10000.0 ** (jnp.arange(0, D, 2, dtype=jnp.float32) / D))
    pos = jnp.arange(SEQ, dtype=jnp.float32)
    freqs = pos[:, None] * inv_freq[None, :]
    cos_t = jnp.repeat(jnp.cos(freqs), 2, axis=1)
    sign = jnp.tile(jnp.array([-1.0, 1.0], jnp.float32), D // 2)
    sin_t = jnp.repeat(jnp.sin(freqs), 2, axis=1) * sign

    mesh = plsc.VectorSubcoreMesh(core_axis_name="c", subcore_axis_name="s")
    f = pl.kernel(
        _sc_body,
        out_type=jax.ShapeDtypeStruct((NB * SEQ, D), jnp.float32),
        mesh=mesh,
        compiler_params=pltpu.CompilerParams(needs_layout_passes=False,
                                             use_tc_tiling_on_sc=False),
        scratch_types=[
            pltpu.VMEM((NCHUNK, CHUNK), jnp.int32),
            pltpu.VMEM((NCHUNK, CHUNK), jnp.int32),
            pltpu.VMEM((SEQ, D), jnp.float32),
            pltpu.VMEM((SEQ, D), jnp.float32),
            pltpu.VMEM((CHUNK, D), jnp.float32),
            pltpu.VMEM((CHUNK, D), jnp.float32),
            pltpu.VMEM((CHUNK, D), jnp.float32),
            pltpu.VMEM((CHUNK, D), jnp.float32),
            pltpu.SemaphoreType.DMA,
            pltpu.SemaphoreType.DMA,
            pltpu.SemaphoreType.DMA,
            pltpu.SemaphoreType.DMA,
        ],
    )
    out = f(x_r, oidx, cos_t, sin_t, table)
    return out.reshape(nb, seq, D)
